# baseline (speedup 1.0000x reference)
import functools

import jax
import jax.numpy as jnp
from jax import lax
from jax.experimental import pallas as pl
from jax.experimental.pallas import tpu as pltpu
from jax.experimental.pallas import tpu_sc as plsc

B, N, S, D1, D2 = 8, 4096, 1024, 128, 256
C1, C2 = 256, 128
BN_ROWS = 512
NB = N // BN_ROWS
M = B * N

NW = 32
PW = M // NW
CH = 16
NCH = PW // CH


def _k1_body(x1_ref, x2t_ref, sq1_ref, sq2_ref, iota_ref, p1_ref,
             w1a_ref, b1_ref, p1out_ref, tw_ref):
    xx = jax.lax.dot_general(x1_ref[...], x2t_ref[...],
                             (((1,), (0,)), ((), ())),
                             preferred_element_type=jnp.float32)
    u = (-2.0 * xx + sq1_ref[...]) + sq2_ref[...]

    m1 = jnp.min(u, axis=1, keepdims=True)
    c1 = u == m1
    d1 = jnp.where(c1, jnp.inf, u)
    m2 = jnp.min(d1, axis=1, keepdims=True)
    c2 = d1 == m2
    d2 = jnp.where(c2, jnp.inf, d1)
    m3 = jnp.min(d2, axis=1, keepdims=True)
    c3 = d2 == m3

    r = [1.0 / (jnp.maximum(v, 0.0) + 1e-8) for v in (m1, m2, m3)]
    norm = r[0] + r[1] + r[2]
    w = [ri / norm for ri in r]

    base = (pl.program_id(0) * S).astype(jnp.float32)
    idx = []
    for ck in (c1, c2, c3):
        ckf = jnp.where(ck, 1.0, 0.0)
        ik = jax.lax.dot_general(ckf, iota_ref[...], (((1,), (0,)), ((), ())),
                                 preferred_element_type=jnp.float32,
                                 precision=jax.lax.Precision.HIGHEST)
        idx.append(jnp.minimum(ik, jnp.float32(S - 1)) + base)

    tw = jnp.concatenate(idx + w + [jnp.zeros((BN_ROWS, 2), jnp.float32)],
                         axis=1)
    tw_ref[...] = tw.T

    p1out_ref[...] = (
        jax.lax.dot_general(p1_ref[...], w1a_ref[...],
                            (((1,), (0,)), ((), ())),
                            preferred_element_type=jnp.float32)
        + b1_ref[...])


def _lane_splat(v, ps):
    return lax.gather(
        v, ps[:, None],
        lax.GatherDimensionNumbers(offset_dims=(), collapsed_slice_dims=(0,),
                                   start_index_map=(0,)),
        (1,), mode=lax.GatherScatterMode.PROMISE_IN_BOUNDS)


def _sc_interp_body(table, idxh0, idxh1, idxh2, wh0, wh1, wh2, out,
                    idx0, idx1, idx2, w0, w1, w2,
                    rows0, rows1, rows2, out_v, sem0, sem1, sem2):
    wid = lax.axis_index("s") * 2 + lax.axis_index("c")
    base = wid * PW
    pltpu.sync_copy(idxh0.at[pl.ds(base, PW)], idx0)
    pltpu.sync_copy(idxh1.at[pl.ds(base, PW)], idx1)
    pltpu.sync_copy(idxh2.at[pl.ds(base, PW)], idx2)
    pltpu.sync_copy(wh0.at[pl.ds(base, PW)], w0)
    pltpu.sync_copy(wh1.at[pl.ds(base, PW)], w1)
    pltpu.sync_copy(wh2.at[pl.ds(base, PW)], w2)

    def chunk(c, carry):
        off = c * CH
        cp0 = pltpu.async_copy(table.at[idx0.at[pl.ds(off, CH)]], rows0, sem0)
        cp1 = pltpu.async_copy(table.at[idx1.at[pl.ds(off, CH)]], rows1, sem1)
        cp2 = pltpu.async_copy(table.at[idx2.at[pl.ds(off, CH)]], rows2, sem2)
        cp0.wait()
        cp1.wait()
        cp2.wait()
        a0 = w0[pl.ds(off, CH)]
        a1 = w1[pl.ds(off, CH)]
        a2 = w2[pl.ds(off, CH)]
        for p in range(CH):
            ps = jnp.full((16,), p, jnp.int32)
            s0 = _lane_splat(a0, ps)
            s1 = _lane_splat(a1, ps)
            s2 = _lane_splat(a2, ps)
            for j in range(D2 // 16):
                sl = pl.ds(j * 16, 16)
                out_v[p, sl] = (rows0[p, sl] * s0 + rows1[p, sl] * s1
                                + rows2[p, sl] * s2)
        pltpu.sync_copy(out_v, out.at[pl.ds(base + off, CH)])
        return carry

    lax.fori_loop(0, NCH, chunk, 0)


def _k2_body(p1out_ref, interp_ref, w1b_ref, h1_ref, ssum_ref, ssq_ref):
    h1 = p1out_ref[...] + jax.lax.dot_general(
        interp_ref[...], w1b_ref[...], (((1,), (0,)), ((), ())),
        preferred_element_type=jnp.float32)
    h1_ref[...] = h1

    @pl.when(pl.program_id(0) == 0)
    def _():
        ssum_ref[...] = jnp.zeros_like(ssum_ref)
        ssq_ref[...] = jnp.zeros_like(ssq_ref)

    ssum_ref[...] += jnp.sum(h1, axis=0, keepdims=True)
    ssq_ref[...] += jnp.sum(h1 * h1, axis=0, keepdims=True)


def _k3_body(h1_ref, sc_ref, sh_ref, w2t_ref, b2_ref,
             h2_ref, ssum_ref, ssq_ref):
    h1n = jnp.maximum(h1_ref[...] * sc_ref[...] + sh_ref[...], 0.0)
    h2 = (jax.lax.dot_general(h1n, w2t_ref[...], (((1,), (0,)), ((), ())),
                              preferred_element_type=jnp.float32)
          + b2_ref[...])
    h2_ref[...] = h2

    @pl.when(pl.program_id(0) == 0)
    def _():
        ssum_ref[...] = jnp.zeros_like(ssum_ref)
        ssq_ref[...] = jnp.zeros_like(ssq_ref)

    ssum_ref[...] += jnp.sum(h2, axis=0, keepdims=True)
    ssq_ref[...] += jnp.sum(h2 * h2, axis=0, keepdims=True)


def _k4_body(h2_ref, sc_ref, sh_ref, out_ref):
    out_ref[...] = jnp.maximum(h2_ref[...] * sc_ref[...] + sh_ref[...], 0.0)


def _affine(ssum, ssq, gamma, beta):
    mean = ssum[0] / M
    var = ssq[0] / M - mean * mean
    scale = gamma * jax.lax.rsqrt(var + 1e-5)
    shift = beta - mean * scale
    return scale[None, :], shift[None, :]


@jax.jit
def kernel(xyz1, xyz2, points1, points2, W1, b1, g1, be1, W2, b2, g2, be2):
    x1f = xyz1.reshape(M, 3)
    x2t = jnp.transpose(xyz2, (0, 2, 1))
    sq1 = jnp.sum(xyz1 ** 2, -1).reshape(M, 1)
    sq2 = jnp.sum(xyz2 ** 2, -1)[:, None, :]
    iota_col = jnp.arange(S, dtype=jnp.float32)[:, None]
    p1f = points1.reshape(M, D1)
    w1a = W1[:, :D1].T
    w1b = W1[:, D1:].T
    w2t = W2.T

    rowblk = lambda r, c: pl.BlockSpec((r, c), lambda b, n: (b * NB + n, 0))
    perb = lambda d0, d1: pl.BlockSpec((None, d0, d1), lambda b, n: (b, 0, 0))
    full = lambda d0, d1: pl.BlockSpec((d0, d1), lambda b, n: (0, 0))

    p1out, tw_t = pl.pallas_call(
        _k1_body,
        grid=(B, NB),
        in_specs=[rowblk(BN_ROWS, 3), perb(3, S), rowblk(BN_ROWS, 1),
                  perb(1, S), full(S, 1), rowblk(BN_ROWS, D1),
                  full(D1, C1), full(1, C1)],
        out_specs=[rowblk(BN_ROWS, C1),
                   pl.BlockSpec((8, BN_ROWS), lambda b, n: (0, b * NB + n))],
        out_shape=[jax.ShapeDtypeStruct((M, C1), jnp.float32),
                   jax.ShapeDtypeStruct((8, M), jnp.float32)],
    )(x1f, x2t, sq1, sq2, iota_col, p1f, w1a, b1[None, :])

    idx_i = tw_t[:3].astype(jnp.int32)
    p2flat = points2.reshape(B * S, D2)

    mesh = plsc.VectorSubcoreMesh(core_axis_name="c", subcore_axis_name="s")
    interp = pl.kernel(
        _sc_interp_body,
        mesh=mesh,
        out_type=jax.ShapeDtypeStruct((M, D2), jnp.float32),
        scratch_types=[
            pltpu.VMEM((PW,), jnp.int32),
            pltpu.VMEM((PW,), jnp.int32),
            pltpu.VMEM((PW,), jnp.int32),
            pltpu.VMEM((PW,), jnp.float32),
            pltpu.VMEM((PW,), jnp.float32),
            pltpu.VMEM((PW,), jnp.float32),
            pltpu.VMEM((CH, D2), jnp.float32),
            pltpu.VMEM((CH, D2), jnp.float32),
            pltpu.VMEM((CH, D2), jnp.float32),
            pltpu.VMEM((CH, D2), jnp.float32),
            pltpu.SemaphoreType.DMA,
            pltpu.SemaphoreType.DMA,
            pltpu.SemaphoreType.DMA,
        ],
    )(p2flat, idx_i[0], idx_i[1], idx_i[2], tw_t[3], tw_t[4], tw_t[5])

    blk = lambda r, c: pl.BlockSpec((r, c), lambda i: (i, 0))
    full1 = lambda d0, d1: pl.BlockSpec((d0, d1), lambda i: (0, 0))

    h1, s1, q1 = pl.pallas_call(
        _k2_body,
        grid=(M // BN_ROWS,),
        in_specs=[blk(BN_ROWS, C1), blk(BN_ROWS, D2), full1(D2, C1)],
        out_specs=[blk(BN_ROWS, C1), full1(1, C1), full1(1, C1)],
        out_shape=[jax.ShapeDtypeStruct((M, C1), jnp.float32),
                   jax.ShapeDtypeStruct((1, C1), jnp.float32),
                   jax.ShapeDtypeStruct((1, C1), jnp.float32)],
    )(p1out, interp, w1b)

    sc1, sh1 = _affine(s1, q1, g1, be1)

    h2, s2, q2 = pl.pallas_call(
        _k3_body,
        grid=(M // BN_ROWS,),
        in_specs=[blk(BN_ROWS, C1), full1(1, C1), full1(1, C1),
                  full1(C1, C2), full1(1, C2)],
        out_specs=[blk(BN_ROWS, C2), full1(1, C2), full1(1, C2)],
        out_shape=[jax.ShapeDtypeStruct((M, C2), jnp.float32),
                   jax.ShapeDtypeStruct((1, C2), jnp.float32),
                   jax.ShapeDtypeStruct((1, C2), jnp.float32)],
    )(h1, sc1, sh1, w2t, b2[None, :])

    sc2, sh2 = _affine(s2, q2, g2, be2)

    out = pl.pallas_call(
        _k4_body,
        grid=(M // BN_ROWS,),
        in_specs=[blk(BN_ROWS, C2), full1(1, C2), full1(1, C2)],
        out_specs=blk(BN_ROWS, C2),
        out_shape=jax.ShapeDtypeStruct((M, C2), jnp.float32),
    )(h2, sc2, sh2)

    return out.reshape(B, N, C2)


# speedup vs baseline: 1.5539x; 1.5539x over previous
import functools

import jax
import jax.numpy as jnp
from jax import lax
from jax.experimental import pallas as pl
from jax.experimental.pallas import tpu as pltpu
from jax.experimental.pallas import tpu_sc as plsc

B, N, S, D1, D2 = 8, 4096, 1024, 128, 256
C1, C2 = 256, 128
BN_ROWS = 512
NB = N // BN_ROWS
M = B * N

NW = 32
PW = M // NW
CH = 16
NCH = PW // CH


def _k1_body(x1_ref, x2t_ref, sq1_ref, sq2_ref, iota_ref, p1_ref,
             w1a_ref, b1_ref, p1out_ref, tw_ref):
    xx = jax.lax.dot_general(x1_ref[...], x2t_ref[...],
                             (((1,), (0,)), ((), ())),
                             preferred_element_type=jnp.float32)
    u = (-2.0 * xx + sq1_ref[...]) + sq2_ref[...]

    m1 = jnp.min(u, axis=1, keepdims=True)
    c1 = u == m1
    d1 = jnp.where(c1, jnp.inf, u)
    m2 = jnp.min(d1, axis=1, keepdims=True)
    c2 = d1 == m2
    d2 = jnp.where(c2, jnp.inf, d1)
    m3 = jnp.min(d2, axis=1, keepdims=True)
    c3 = d2 == m3

    r = [1.0 / (jnp.maximum(v, 0.0) + 1e-8) for v in (m1, m2, m3)]
    norm = r[0] + r[1] + r[2]
    w = [ri / norm for ri in r]

    base = (pl.program_id(0) * S).astype(jnp.float32)
    idx = []
    for ck in (c1, c2, c3):
        ckf = jnp.where(ck, 1.0, 0.0)
        ik2 = jax.lax.dot_general(ckf, iota_ref[...], (((1,), (0,)), ((), ())),
                                  preferred_element_type=jnp.float32)
        ik = ik2[:, 0:1] * 64.0 + ik2[:, 1:2]
        idx.append(jnp.minimum(ik, jnp.float32(S - 1)) + base)

    tw = jnp.concatenate(idx + w + [jnp.zeros((BN_ROWS, 2), jnp.float32)],
                         axis=1)
    tw_ref[...] = tw.T

    p1out_ref[...] = (
        jax.lax.dot_general(p1_ref[...], w1a_ref[...],
                            (((1,), (0,)), ((), ())),
                            preferred_element_type=jnp.float32)
        + b1_ref[...])


def _lane_splat(v, ps):
    return lax.gather(
        v, ps[:, None],
        lax.GatherDimensionNumbers(offset_dims=(), collapsed_slice_dims=(0,),
                                   start_index_map=(0,)),
        (1,), mode=lax.GatherScatterMode.PROMISE_IN_BOUNDS)


def _sc_interp_body(table, idxh0, idxh1, idxh2, wh0, wh1, wh2, out,
                    idx0, idx1, idx2, w0, w1, w2,
                    r0a, r1a, r2a, r0b, r1b, r2b, out_va, out_vb,
                    s0a, s1a, s2a, s0b, s1b, s2b):
    wid = lax.axis_index("s") * 2 + lax.axis_index("c")
    base = wid * PW
    pltpu.sync_copy(idxh0.at[pl.ds(base, PW)], idx0)
    pltpu.sync_copy(idxh1.at[pl.ds(base, PW)], idx1)
    pltpu.sync_copy(idxh2.at[pl.ds(base, PW)], idx2)
    pltpu.sync_copy(wh0.at[pl.ds(base, PW)], w0)
    pltpu.sync_copy(wh1.at[pl.ds(base, PW)], w1)
    pltpu.sync_copy(wh2.at[pl.ds(base, PW)], w2)

    def _gather(rows, sems, off):
        pltpu.async_copy(table.at[idx0.at[pl.ds(off, CH)]], rows[0], sems[0])
        pltpu.async_copy(table.at[idx1.at[pl.ds(off, CH)]], rows[1], sems[1])
        pltpu.async_copy(table.at[idx2.at[pl.ds(off, CH)]], rows[2], sems[2])

    def _wait(rows, sems):
        for r, s in zip(rows, sems):
            pltpu.make_async_copy(table.at[pl.ds(0, CH)], r, s).wait()

    def _compute(rows, ov, off):
        a0 = w0[pl.ds(off, CH)]
        a1 = w1[pl.ds(off, CH)]
        a2 = w2[pl.ds(off, CH)]
        for p in range(CH):
            ps = jnp.full((16,), p, jnp.int32)
            s0 = _lane_splat(a0, ps)
            s1 = _lane_splat(a1, ps)
            s2 = _lane_splat(a2, ps)
            for j in range(D2 // 16):
                sl = pl.ds(j * 16, 16)
                ov[p, sl] = (rows[0][p, sl] * s0 + rows[1][p, sl] * s1
                             + rows[2][p, sl] * s2)
        pltpu.sync_copy(ov, out.at[pl.ds(base + off, CH)])

    bufa = (r0a, r1a, r2a)
    bufb = (r0b, r1b, r2b)
    sema = (s0a, s1a, s2a)
    semb = (s0b, s1b, s2b)

    _gather(bufa, sema, 0)

    def super_chunk(si, carry):
        off0 = si * (2 * CH)
        off1 = off0 + CH
        off2 = jnp.minimum(off0 + 2 * CH, PW - CH)
        _gather(bufb, semb, off1)
        _wait(bufa, sema)
        _compute(bufa, out_va, off0)
        _gather(bufa, sema, off2)
        _wait(bufb, semb)
        _compute(bufb, out_vb, off1)
        return carry

    lax.fori_loop(0, NCH // 2, super_chunk, 0)
    _wait(bufa, sema)


def _k2_body(p1out_ref, interp_ref, w1b_ref, h1_ref, ssum_ref, ssq_ref):
    h1 = p1out_ref[...] + jax.lax.dot_general(
        interp_ref[...], w1b_ref[...], (((1,), (0,)), ((), ())),
        preferred_element_type=jnp.float32)
    h1_ref[...] = h1

    @pl.when(pl.program_id(0) == 0)
    def _():
        ssum_ref[...] = jnp.zeros_like(ssum_ref)
        ssq_ref[...] = jnp.zeros_like(ssq_ref)

    ssum_ref[...] += jnp.sum(h1, axis=0, keepdims=True)
    ssq_ref[...] += jnp.sum(h1 * h1, axis=0, keepdims=True)


def _k3_body(h1_ref, sc_ref, sh_ref, w2t_ref, b2_ref,
             h2_ref, ssum_ref, ssq_ref):
    h1n = jnp.maximum(h1_ref[...] * sc_ref[...] + sh_ref[...], 0.0)
    h2 = (jax.lax.dot_general(h1n, w2t_ref[...], (((1,), (0,)), ((), ())),
                              preferred_element_type=jnp.float32)
          + b2_ref[...])
    h2_ref[...] = h2

    @pl.when(pl.program_id(0) == 0)
    def _():
        ssum_ref[...] = jnp.zeros_like(ssum_ref)
        ssq_ref[...] = jnp.zeros_like(ssq_ref)

    ssum_ref[...] += jnp.sum(h2, axis=0, keepdims=True)
    ssq_ref[...] += jnp.sum(h2 * h2, axis=0, keepdims=True)


def _k4_body(h2_ref, sc_ref, sh_ref, out_ref):
    out_ref[...] = jnp.maximum(h2_ref[...] * sc_ref[...] + sh_ref[...], 0.0)


def _affine(ssum, ssq, gamma, beta):
    mean = ssum[0] / M
    var = ssq[0] / M - mean * mean
    scale = gamma * jax.lax.rsqrt(var + 1e-5)
    shift = beta - mean * scale
    return scale[None, :], shift[None, :]


@jax.jit
def kernel(xyz1, xyz2, points1, points2, W1, b1, g1, be1, W2, b2, g2, be2):
    x1f = xyz1.reshape(M, 3)
    x2t = jnp.transpose(xyz2, (0, 2, 1))
    sq1 = jnp.sum(xyz1 ** 2, -1).reshape(M, 1)
    sq2 = jnp.sum(xyz2 ** 2, -1)[:, None, :]
    sarange = jnp.arange(S, dtype=jnp.float32)
    iota_col = jnp.stack([jnp.floor(sarange / 64.0), sarange % 64.0],
                         axis=1)
    p1f = points1.reshape(M, D1)
    w1a = W1[:, :D1].T
    w1b = W1[:, D1:].T
    w2t = W2.T

    rowblk = lambda r, c: pl.BlockSpec((r, c), lambda b, n: (b * NB + n, 0))
    perb = lambda d0, d1: pl.BlockSpec((None, d0, d1), lambda b, n: (b, 0, 0))
    full = lambda d0, d1: pl.BlockSpec((d0, d1), lambda b, n: (0, 0))

    p1out, tw_t = pl.pallas_call(
        _k1_body,
        grid=(B, NB),
        in_specs=[rowblk(BN_ROWS, 3), perb(3, S), rowblk(BN_ROWS, 1),
                  perb(1, S), full(S, 2), rowblk(BN_ROWS, D1),
                  full(D1, C1), full(1, C1)],
        out_specs=[rowblk(BN_ROWS, C1),
                   pl.BlockSpec((8, BN_ROWS), lambda b, n: (0, b * NB + n))],
        out_shape=[jax.ShapeDtypeStruct((M, C1), jnp.float32),
                   jax.ShapeDtypeStruct((8, M), jnp.float32)],
    )(x1f, x2t, sq1, sq2, iota_col, p1f, w1a, b1[None, :])

    idx_i = tw_t[:3].astype(jnp.int32)
    p2flat = points2.reshape(B * S, D2)

    mesh = plsc.VectorSubcoreMesh(core_axis_name="c", subcore_axis_name="s")
    interp = pl.kernel(
        _sc_interp_body,
        mesh=mesh,
        out_type=jax.ShapeDtypeStruct((M, D2), jnp.float32),
        scratch_types=(
            [pltpu.VMEM((PW,), jnp.int32)] * 3
            + [pltpu.VMEM((PW,), jnp.float32)] * 3
            + [pltpu.VMEM((CH, D2), jnp.float32)] * 8
            + [pltpu.SemaphoreType.DMA] * 6
        ),
    )(p2flat, idx_i[0], idx_i[1], idx_i[2], tw_t[3], tw_t[4], tw_t[5])

    blk = lambda r, c: pl.BlockSpec((r, c), lambda i: (i, 0))
    full1 = lambda d0, d1: pl.BlockSpec((d0, d1), lambda i: (0, 0))

    h1, s1, q1 = pl.pallas_call(
        _k2_body,
        grid=(M // BN_ROWS,),
        in_specs=[blk(BN_ROWS, C1), blk(BN_ROWS, D2), full1(D2, C1)],
        out_specs=[blk(BN_ROWS, C1), full1(1, C1), full1(1, C1)],
        out_shape=[jax.ShapeDtypeStruct((M, C1), jnp.float32),
                   jax.ShapeDtypeStruct((1, C1), jnp.float32),
                   jax.ShapeDtypeStruct((1, C1), jnp.float32)],
    )(p1out, interp, w1b)

    sc1, sh1 = _affine(s1, q1, g1, be1)

    h2, s2, q2 = pl.pallas_call(
        _k3_body,
        grid=(M // BN_ROWS,),
        in_specs=[blk(BN_ROWS, C1), full1(1, C1), full1(1, C1),
                  full1(C1, C2), full1(1, C2)],
        out_specs=[blk(BN_ROWS, C2), full1(1, C2), full1(1, C2)],
        out_shape=[jax.ShapeDtypeStruct((M, C2), jnp.float32),
                   jax.ShapeDtypeStruct((1, C2), jnp.float32),
                   jax.ShapeDtypeStruct((1, C2), jnp.float32)],
    )(h1, sc1, sh1, w2t, b2[None, :])

    sc2, sh2 = _affine(s2, q2, g2, be2)

    out = pl.pallas_call(
        _k4_body,
        grid=(M // BN_ROWS,),
        in_specs=[blk(BN_ROWS, C2), full1(1, C2), full1(1, C2)],
        out_specs=blk(BN_ROWS, C2),
        out_shape=jax.ShapeDtypeStruct((M, C2), jnp.float32),
    )(h2, sc2, sh2)

    return out.reshape(B, N, C2)
